# baseline (speedup 1.0000x reference)
import functools
import math

import jax
import jax.numpy as jnp
from jax import lax
from jax.experimental import pallas as pl
from jax.experimental.pallas import tpu as pltpu
from jax.experimental.pallas import tpu_sc as plsc

NUM_HEADS = 16
NUM_BUCKETS = 32
MAX_DISTANCE = 128
Q_LEN = 2048
K_LEN = 2048

LINE_PAD = 4224
VAR_W = 3968
NC = 2
NS = 16
HEADS_PER_SC = NUM_HEADS // NC
GROUPS = Q_LEN // 8
K_PER_TEC = GROUPS // NS


def _line_kernel(table_ref, delta_ref, out_ref):
    shape = (1, 1, LINE_PAD)
    x = lax.broadcasted_iota(jnp.int32, shape, 2)
    rel = x - (K_LEN - 1) + delta_ref[0, 0]
    n = -rel
    sign = jnp.where(n > 0, NUM_BUCKETS // 2, 0)
    na = jnp.abs(n)
    half = NUM_BUCKETS // 2
    is_small = na < half
    nc = jnp.maximum(na, 1).astype(jnp.float32)
    log_ratio = jnp.log(nc / half) / math.log(MAX_DISTANCE / half)
    vl = jnp.floor(log_ratio * (NUM_BUCKETS - half)).astype(jnp.int32) + half
    vl = jnp.minimum(vl, NUM_BUCKETS - 1)
    val = jnp.where(is_small, na, vl)
    b = jnp.clip(val + sign, 0, NUM_BUCKETS - 1)
    line = jnp.zeros(shape, jnp.float32)
    for t in range(NUM_BUCKETS):
        line = jnp.where(b == t, table_ref[0, 0, t], line)
    for r in range(8):
        out_ref[0, r, :] = lax.slice(line, (0, 0, r), (1, 1, r + LINE_PAD - 8))[0, 0]


def _compute_lines8(table, delta):
    table_t3 = jnp.transpose(table).reshape(NUM_HEADS, 1, NUM_BUCKETS)
    return pl.pallas_call(
        _line_kernel,
        grid=(NUM_HEADS,),
        in_specs=[
            pl.BlockSpec((1, 1, NUM_BUCKETS), lambda h: (h, 0, 0)),
            pl.BlockSpec(memory_space=pltpu.SMEM),
        ],
        out_specs=pl.BlockSpec((1, 8, LINE_PAD - 8), lambda h: (h, 0, 0)),
        out_shape=jax.ShapeDtypeStruct((NUM_HEADS, 8, LINE_PAD - 8), jnp.float32),
    )(table_t3, delta)


def _sc_body(lines8_hbm, out_hbm, var_a, var_b, load_sem, write_sem):
    c = lax.axis_index("c")
    t = lax.axis_index("s")
    bufs = (var_a, var_b)
    q = pl.multiple_of(120 - 8 * t, 8)

    def load_start(hh, buf):
        h = c * HEADS_PER_SC + hh
        for r8 in range(8):
            pltpu.make_async_copy(
                lines8_hbm.at[h, 7 - r8, pl.ds(q, VAR_W)],
                buf.at[r8],
                load_sem,
            ).start()

    def load_wait(buf):
        for r8 in range(8):
            pltpu.make_async_copy(
                lines8_hbm.at[0, 0, pl.ds(0, VAR_W)], buf.at[0], load_sem
            ).wait()

    def drain_writes():
        for _ in range(K_PER_TEC):
            pltpu.make_async_copy(
                bufs[0].at[:, pl.ds(0, K_LEN)],
                out_hbm.at[0, 0, pl.ds(0, 8), :],
                write_sem,
            ).wait()

    load_start(0, bufs[0])
    for hh in range(HEADS_PER_SC):
        buf = bufs[hh % 2]
        load_wait(buf)
        h = c * HEADS_PER_SC + hh
        for k in range(K_PER_TEC):
            p0 = (K_PER_TEC - 1 - k) * 128
            i8 = pl.multiple_of(128 * k + 8 * t, 8)
            pltpu.make_async_copy(
                buf.at[:, pl.ds(p0, K_LEN)],
                out_hbm.at[0, h, pl.ds(i8, 8), :],
                write_sem,
            ).start()
        if hh >= 1:
            drain_writes()
        if hh + 1 < HEADS_PER_SC:
            load_start(hh + 1, bufs[(hh + 1) % 2])
    drain_writes()


_MATERIALIZE_CACHE = []


def _materialize_fn():
    if not _MATERIALIZE_CACHE:
        _MATERIALIZE_CACHE.append(functools.partial(
            pl.kernel,
            out_type=jax.ShapeDtypeStruct((1, NUM_HEADS, Q_LEN, K_LEN), jnp.float32),
            mesh=plsc.VectorSubcoreMesh(
                core_axis_name="c", subcore_axis_name="s",
                num_cores=NC, num_subcores=NS,
            ),
            compiler_params=pltpu.CompilerParams(use_tc_tiling_on_sc=False),
            scratch_types=[
                pltpu.VMEM((8, VAR_W), jnp.float32),
                pltpu.VMEM((8, VAR_W), jnp.float32),
                pltpu.SemaphoreType.DMA,
                pltpu.SemaphoreType.DMA,
            ],
        )(_sc_body))
    return _MATERIALIZE_CACHE[0]


def kernel(q_len, k_len, relative_attention_bias):
    q_res = jnp.asarray(q_len, jnp.int32) - Q_LEN
    k_res = jnp.asarray(k_len, jnp.int32) - K_LEN
    delta = (k_res - q_res).reshape(1, 1)
    lines8 = _compute_lines8(relative_attention_bias, delta)
    return _materialize_fn()(lines8)


# speedup vs baseline: 3.0192x; 3.0192x over previous
import functools
import math

import jax
import jax.numpy as jnp
from jax import lax
from jax.experimental import pallas as pl
from jax.experimental.pallas import tpu as pltpu
from jax.experimental.pallas import tpu_sc as plsc

NUM_HEADS = 16
NUM_BUCKETS = 32
MAX_DISTANCE = 128
Q_LEN = 2048
K_LEN = 2048

LINE_PAD = 4224
VAR_W = 3968
NC = 2
NS = 16
HEADS_PER_SC = NUM_HEADS // NC
GROUPS = Q_LEN // 8
K_PER_TEC = GROUPS // NS


def _line_kernel(table_ref, delta_ref, out_ref):
    shape = (1, 1, LINE_PAD)
    x = lax.broadcasted_iota(jnp.int32, shape, 2)
    rel = x - (K_LEN - 1) + delta_ref[0, 0]
    n = -rel
    sign = jnp.where(n > 0, NUM_BUCKETS // 2, 0)
    na = jnp.abs(n)
    half = NUM_BUCKETS // 2
    is_small = na < half
    nc = jnp.maximum(na, 1).astype(jnp.float32)
    log_ratio = jnp.log(nc / half) / math.log(MAX_DISTANCE / half)
    vl = jnp.floor(log_ratio * (NUM_BUCKETS - half)).astype(jnp.int32) + half
    vl = jnp.minimum(vl, NUM_BUCKETS - 1)
    val = jnp.where(is_small, na, vl)
    b = jnp.clip(val + sign, 0, NUM_BUCKETS - 1)
    line = jnp.zeros(shape, jnp.float32)
    for t in range(NUM_BUCKETS):
        line = jnp.where(b == t, table_ref[0, 0, t], line)
    for t in range(NS):
        for r8 in range(8):
            s = 127 - 8 * t - r8
            out_ref[0, t, r8, :] = lax.slice(line, (0, 0, s), (1, 1, s + VAR_W))[0, 0]


def _compute_linev8(table, delta):
    table_t3 = jnp.transpose(table).reshape(NUM_HEADS, 1, NUM_BUCKETS)
    return pl.pallas_call(
        _line_kernel,
        grid=(NUM_HEADS,),
        in_specs=[
            pl.BlockSpec((1, 1, NUM_BUCKETS), lambda h: (h, 0, 0)),
            pl.BlockSpec(memory_space=pltpu.SMEM),
        ],
        out_specs=pl.BlockSpec((1, NS, 8, VAR_W), lambda h: (h, 0, 0, 0)),
        out_shape=jax.ShapeDtypeStruct((NUM_HEADS, NS, 8, VAR_W), jnp.float32),
    )(table_t3, delta)


def _sc_body(linev8_hbm, out_hbm, var_a, var_b, load_sem, write_sem):
    c = lax.axis_index("c")
    t = lax.axis_index("s")
    bufs = (var_a, var_b)

    def load(hh, buf):
        h = c * HEADS_PER_SC + hh
        return pltpu.make_async_copy(linev8_hbm.at[h, t], buf, load_sem)

    def drain_writes():
        for _ in range(K_PER_TEC):
            pltpu.make_async_copy(
                bufs[0].at[:, pl.ds(0, K_LEN)],
                out_hbm.at[0, 0, pl.ds(0, 8), :],
                write_sem,
            ).wait()

    load(0, bufs[0]).start()
    for hh in range(HEADS_PER_SC):
        buf = bufs[hh % 2]
        load(hh, buf).wait()
        h = c * HEADS_PER_SC + hh
        for k in range(K_PER_TEC):
            p0 = (K_PER_TEC - 1 - k) * 128
            i8 = pl.multiple_of(128 * k + 8 * t, 8)
            pltpu.make_async_copy(
                buf.at[:, pl.ds(p0, K_LEN)],
                out_hbm.at[0, h, pl.ds(i8, 8), :],
                write_sem,
            ).start()
        if hh >= 1:
            drain_writes()
        if hh + 1 < HEADS_PER_SC:
            load(hh + 1, bufs[(hh + 1) % 2]).start()
    drain_writes()


_MATERIALIZE_CACHE = []


def _materialize_fn():
    if not _MATERIALIZE_CACHE:
        _MATERIALIZE_CACHE.append(functools.partial(
            pl.kernel,
            out_type=jax.ShapeDtypeStruct((1, NUM_HEADS, Q_LEN, K_LEN), jnp.float32),
            mesh=plsc.VectorSubcoreMesh(
                core_axis_name="c", subcore_axis_name="s",
                num_cores=NC, num_subcores=NS,
            ),
            scratch_types=[
                pltpu.VMEM((8, VAR_W), jnp.float32),
                pltpu.VMEM((8, VAR_W), jnp.float32),
                pltpu.SemaphoreType.DMA,
                pltpu.SemaphoreType.DMA,
            ],
        )(_sc_body))
    return _MATERIALIZE_CACHE[0]


def kernel(q_len, k_len, relative_attention_bias):
    q_res = jnp.asarray(q_len, jnp.int32) - Q_LEN
    k_res = jnp.asarray(k_len, jnp.int32) - K_LEN
    delta = (k_res - q_res).reshape(1, 1)
    linev8 = _compute_linev8(relative_attention_bias, delta)
    return _materialize_fn()(linev8)


# speedup vs baseline: 3.0891x; 1.0231x over previous
import functools
import math

import jax
import jax.numpy as jnp
from jax import lax
from jax.experimental import pallas as pl
from jax.experimental.pallas import tpu as pltpu
from jax.experimental.pallas import tpu_sc as plsc

NUM_HEADS = 16
NUM_BUCKETS = 32
MAX_DISTANCE = 128
Q_LEN = 2048
K_LEN = 2048

LINE_PAD = 4224
VAR_W = 3968
NC = 2
NS = 16
HEADS_PER_SC = NUM_HEADS // NC
GROUPS = Q_LEN // 8
K_PER_TEC = GROUPS // NS


def _line_kernel(table_ref, delta_ref, out_ref):
    shape = (1, 1, LINE_PAD)
    x = lax.broadcasted_iota(jnp.int32, shape, 2)
    rel = x - (K_LEN - 1) + delta_ref[0, 0]
    n = -rel
    sign = jnp.where(n > 0, NUM_BUCKETS // 2, 0)
    na = jnp.abs(n)
    half = NUM_BUCKETS // 2
    is_small = na < half
    nc = jnp.maximum(na, 1).astype(jnp.float32)
    log_ratio = jnp.log(nc / half) / math.log(MAX_DISTANCE / half)
    vl = jnp.floor(log_ratio * (NUM_BUCKETS - half)).astype(jnp.int32) + half
    vl = jnp.minimum(vl, NUM_BUCKETS - 1)
    val = jnp.where(is_small, na, vl)
    b = jnp.clip(val + sign, 0, NUM_BUCKETS - 1)
    line = jnp.zeros(shape, jnp.float32)
    for t in range(NUM_BUCKETS):
        line = jnp.where(b == t, table_ref[0, 0, t], line)
    for t in range(NS):
        for r8 in range(8):
            s = 127 - 8 * t - r8
            out_ref[0, t, r8, :] = lax.slice(line, (0, 0, s), (1, 1, s + VAR_W))[0, 0]


def _compute_linev8(table, delta):
    table_t3 = jnp.transpose(table).reshape(NUM_HEADS, 1, NUM_BUCKETS)
    return pl.pallas_call(
        _line_kernel,
        grid=(NUM_HEADS,),
        in_specs=[
            pl.BlockSpec((1, 1, NUM_BUCKETS), lambda h: (h, 0, 0)),
            pl.BlockSpec(memory_space=pltpu.SMEM),
        ],
        out_specs=pl.BlockSpec((1, NS, 8, VAR_W), lambda h: (h, 0, 0, 0)),
        out_shape=jax.ShapeDtypeStruct((NUM_HEADS, NS, 8, VAR_W), jnp.float32),
    )(table_t3, delta)


def _sc_body(linev8_hbm, out_hbm, var_a, var_b, var_c, load_sem, write_sem):
    c = lax.axis_index("c")
    t = lax.axis_index("s")
    bufs = (var_a, var_b, var_c)

    def load(hh, buf):
        h = c + NC * hh
        return pltpu.make_async_copy(linev8_hbm.at[h, t], buf, load_sem)

    def drain_writes():
        for _ in range(K_PER_TEC):
            pltpu.make_async_copy(
                bufs[0].at[:, pl.ds(0, K_LEN)],
                out_hbm.at[0, 0, pl.ds(0, 8), :],
                write_sem,
            ).wait()

    load(0, bufs[0]).start()
    load(1, bufs[1]).start()
    for hh in range(HEADS_PER_SC):
        buf = bufs[hh % 3]
        load(hh, buf).wait()
        h = c + NC * hh
        for k in range(K_PER_TEC):
            p0 = (K_PER_TEC - 1 - k) * 128
            i8 = pl.multiple_of(128 * k + 8 * t, 8)
            pltpu.make_async_copy(
                buf.at[:, pl.ds(p0, K_LEN)],
                out_hbm.at[0, h, pl.ds(i8, 8), :],
                write_sem,
            ).start()
        if hh >= 1:
            drain_writes()
        if hh + 2 < HEADS_PER_SC:
            load(hh + 2, bufs[(hh + 2) % 3]).start()
    drain_writes()


_MATERIALIZE_CACHE = []


def _materialize_fn():
    if not _MATERIALIZE_CACHE:
        _MATERIALIZE_CACHE.append(functools.partial(
            pl.kernel,
            out_type=jax.ShapeDtypeStruct((1, NUM_HEADS, Q_LEN, K_LEN), jnp.float32),
            mesh=plsc.VectorSubcoreMesh(
                core_axis_name="c", subcore_axis_name="s",
                num_cores=NC, num_subcores=NS,
            ),
            scratch_types=[
                pltpu.VMEM((8, VAR_W), jnp.float32),
                pltpu.VMEM((8, VAR_W), jnp.float32),
                pltpu.VMEM((8, VAR_W), jnp.float32),
                pltpu.SemaphoreType.DMA,
                pltpu.SemaphoreType.DMA,
            ],
        )(_sc_body))
    return _MATERIALIZE_CACHE[0]


def kernel(q_len, k_len, relative_attention_bias):
    q_res = jnp.asarray(q_len, jnp.int32) - Q_LEN
    k_res = jnp.asarray(k_len, jnp.int32) - K_LEN
    delta = (k_res - q_res).reshape(1, 1)
    linev8 = _compute_linev8(relative_attention_bias, delta)
    return _materialize_fn()(linev8)


# speedup vs baseline: 3.1559x; 1.0216x over previous
import functools
import math

import jax
import jax.numpy as jnp
from jax import lax
from jax.experimental import pallas as pl
from jax.experimental.pallas import tpu as pltpu
from jax.experimental.pallas import tpu_sc as plsc

NUM_HEADS = 16
NUM_BUCKETS = 32
MAX_DISTANCE = 128
Q_LEN = 2048
K_LEN = 2048

LINE_PAD = 4224
VAR_W = 3968
NC = 2
NS = 16
HEADS_PER_SC = NUM_HEADS // NC
GROUPS = Q_LEN // 8
K_PER_TEC = GROUPS // NS
LANES = 16


def _line_kernel(table_ref, delta_ref, out_ref):
    shape = (1, 1, LINE_PAD)
    x = lax.broadcasted_iota(jnp.int32, shape, 2)
    rel = x - (K_LEN - 1) + delta_ref[0, 0]
    n = -rel
    sign = jnp.where(n > 0, NUM_BUCKETS // 2, 0)
    na = jnp.abs(n)
    half = NUM_BUCKETS // 2
    is_small = na < half
    nc = jnp.maximum(na, 1).astype(jnp.float32)
    log_ratio = jnp.log(nc / half) / math.log(MAX_DISTANCE / half)
    vl = jnp.floor(log_ratio * (NUM_BUCKETS - half)).astype(jnp.int32) + half
    vl = jnp.minimum(vl, NUM_BUCKETS - 1)
    val = jnp.where(is_small, na, vl)
    b = jnp.clip(val + sign, 0, NUM_BUCKETS - 1)
    line = jnp.zeros(shape, jnp.float32)
    for t in range(NUM_BUCKETS):
        line = jnp.where(b == t, table_ref[0, 0, t], line)
    out_ref[...] = line


def _compute_lines(table, delta):
    table_t3 = jnp.transpose(table).reshape(NUM_HEADS, 1, NUM_BUCKETS)
    return pl.pallas_call(
        _line_kernel,
        grid=(NUM_HEADS,),
        in_specs=[
            pl.BlockSpec((1, 1, NUM_BUCKETS), lambda h: (h, 0, 0)),
            pl.BlockSpec(memory_space=pltpu.SMEM),
        ],
        out_specs=pl.BlockSpec((1, 1, LINE_PAD), lambda h: (h, 0, 0)),
        out_shape=jax.ShapeDtypeStruct((NUM_HEADS, 1, LINE_PAD), jnp.float32),
    )(table_t3, delta)


def _sc_body(lines_hbm, out_hbm, line_a, line_b, var_a, var_b, var_c,
             line_sem, write_sem):
    c = lax.axis_index("c")
    t = lax.axis_index("s")
    vbufs = (var_a, var_b, var_c)
    lbufs = (line_a, line_b)
    iota16 = lax.broadcasted_iota(jnp.int32, (LANES,), 0)

    def line_load(hh, lbuf):
        h = c + NC * hh
        return pltpu.make_async_copy(lines_hbm.at[h, 0, :], lbuf, line_sem)

    def assemble(lbuf, vbuf):
        for r8 in range(8):
            s = 127 - 8 * t - r8
            base = s + iota16

            def body(i, _):
                cc = i * (8 * LANES)
                for u in range(8):
                    col = pl.multiple_of(cc + u * LANES, LANES)
                    vbuf[r8, pl.ds(col, LANES)] = plsc.load_gather(
                        lbuf, [col + base])
                return 0

            lax.fori_loop(0, VAR_W // (8 * LANES), body, 0)

    def drain_writes():
        for _ in range(K_PER_TEC):
            pltpu.make_async_copy(
                vbufs[0].at[:, pl.ds(0, K_LEN)],
                out_hbm.at[0, 0, pl.ds(0, 8), :],
                write_sem,
            ).wait()

    line_load(0, lbufs[0]).start()
    line_load(1, lbufs[1]).start()
    for hh in range(HEADS_PER_SC):
        lbuf = lbufs[hh % 2]
        vbuf = vbufs[hh % 3]
        line_load(hh, lbuf).wait()
        assemble(lbuf, vbuf)
        if hh + 2 < HEADS_PER_SC:
            line_load(hh + 2, lbufs[hh % 2]).start()
        h = c + NC * hh
        for k in range(K_PER_TEC):
            p0 = (K_PER_TEC - 1 - k) * 128
            i8 = pl.multiple_of(128 * k + 8 * t, 8)
            pltpu.make_async_copy(
                vbuf.at[:, pl.ds(p0, K_LEN)],
                out_hbm.at[0, h, pl.ds(i8, 8), :],
                write_sem,
            ).start()
        if hh >= 1:
            drain_writes()
    drain_writes()


_MATERIALIZE_CACHE = []


def _materialize_fn():
    if not _MATERIALIZE_CACHE:
        _MATERIALIZE_CACHE.append(functools.partial(
            pl.kernel,
            out_type=jax.ShapeDtypeStruct((1, NUM_HEADS, Q_LEN, K_LEN), jnp.float32),
            mesh=plsc.VectorSubcoreMesh(
                core_axis_name="c", subcore_axis_name="s",
                num_cores=NC, num_subcores=NS,
            ),
            compiler_params=pltpu.CompilerParams(needs_layout_passes=False),
            scratch_types=[
                pltpu.VMEM((LINE_PAD,), jnp.float32),
                pltpu.VMEM((LINE_PAD,), jnp.float32),
                pltpu.VMEM((8, VAR_W), jnp.float32),
                pltpu.VMEM((8, VAR_W), jnp.float32),
                pltpu.VMEM((8, VAR_W), jnp.float32),
                pltpu.SemaphoreType.DMA,
                pltpu.SemaphoreType.DMA,
            ],
        )(_sc_body))
    return _MATERIALIZE_CACHE[0]


def kernel(q_len, k_len, relative_attention_bias):
    q_res = jnp.asarray(q_len, jnp.int32) - Q_LEN
    k_res = jnp.asarray(k_len, jnp.int32) - K_LEN
    delta = (k_res - q_res).reshape(1, 1)
    lines = _compute_lines(relative_attention_bias, delta)
    return _materialize_fn()(lines)
